# Initial kernel scaffold; baseline (speedup 1.0000x reference)
import jax
import jax.numpy as jnp
from jax.experimental import pallas as pl


def kernel(x, edge_index):
    raise NotImplementedError("write your pallas kernel here")


# speedup vs baseline: 26.7335x; 26.7335x over previous
import dataclasses
import functools

import jax
import jax.numpy as jnp
from jax import lax
from jax.experimental import pallas as pl
from jax.experimental.pallas import tpu as pltpu
from jax.experimental.pallas import tpu_sc as plsc

N = 10000
D = 128
E = 320000

NC = 2
NS = 16
NW = NC * NS
LANES = 16

CHUNK = 128
NCHUNKS = -(-E // CHUNK)
CH_PER_TILE = ((-(-NCHUNKS // NW)) + 7) // 8 * 8
NCHUNKS_P = CH_PER_TILE * NW
E_PAD = NCHUNKS_P * CHUNK
N_PAD = 10240
RB = 2
IDXB = 16

_mesh = plsc.VectorSubcoreMesh(core_axis_name="c", subcore_axis_name="s")

_cp = pltpu.CompilerParams()
if "needs_layout_passes" in pltpu.CompilerParams.__dataclass_fields__:
    _cp = dataclasses.replace(_cp, needs_layout_passes=False)


@functools.partial(
    pl.kernel,
    out_type=jax.ShapeDtypeStruct((NW * N_PAD,), jnp.float32),
    mesh=_mesh,
    compiler_params=_cp,
    scratch_types=[
        pltpu.VMEM((CH_PER_TILE, CHUNK), jnp.int32),
        pltpu.VMEM((N_PAD,), jnp.float32),
    ],
)
def _sc_degree(row_hbm, zero_hbm, hist_hbm, ridx_b, hist_v):
    cid = lax.axis_index("c")
    sid = lax.axis_index("s")
    wid = cid * NS + sid

    pltpu.sync_copy(row_hbm.at[pl.ds(wid * CH_PER_TILE, CH_PER_TILE)], ridx_b)
    pltpu.sync_copy(zero_hbm, hist_v)

    @pl.loop(0, CH_PER_TILE)
    def _(c):
        for k in range(0, CHUNK, LANES):
            idx16 = ridx_b[c, pl.ds(k, LANES)]
            plsc.addupdate_scatter(
                hist_v, [idx16], jnp.ones((LANES,), jnp.float32)
            )

    pltpu.sync_copy(hist_v, hist_hbm.at[pl.ds(wid * N_PAD, N_PAD)])


@functools.partial(
    pl.kernel,
    out_type=jax.ShapeDtypeStruct((NC * N_PAD, D), jnp.float32),
    mesh=_mesh,
    scratch_types=[
        pltpu.VMEM((IDXB, CHUNK), jnp.int32),
        pltpu.VMEM((IDXB, CHUNK), jnp.int32),
        pltpu.VMEM_SHARED((N_PAD, D), jnp.float32),
    ]
    + [pltpu.VMEM((CHUNK, D), jnp.float32)] * RB
    + [pltpu.SemaphoreType.DMA] * (2 * RB),
)
def _sc_aggregate(
    y_hbm, row_hbm, col_hbm, zero_hbm, acc_hbm,
    ridx_b, cidx_b, acc_sh, *bufs_sems,
):
    rbufs = bufs_sems[:RB]
    gsems = bufs_sems[RB:2 * RB]
    ssems = bufs_sems[2 * RB:]
    cid = lax.axis_index("c")
    sid = lax.axis_index("s")
    wid = cid * NS + sid
    base = wid * CH_PER_TILE

    @pl.when(sid == 0)
    def _():
        pltpu.sync_copy(zero_hbm, acc_sh)

    plsc.subcore_barrier()

    @pl.loop(0, CH_PER_TILE // IDXB)
    def _(h):
        hbase = base + h * IDXB
        pltpu.sync_copy(row_hbm.at[pl.ds(hbase, IDXB)], ridx_b)
        pltpu.sync_copy(col_hbm.at[pl.ds(hbase, IDXB)], cidx_b)

        @pl.loop(0, IDXB // RB)
        def _(p):
            c0 = p * RB
            ghs = [
                pltpu.async_copy(y_hbm.at[cidx_b.at[c0 + i]], rbufs[i], gsems[i])
                for i in range(RB)
            ]
            shs = []
            for i in range(RB):
                ghs[i].wait()
                shs.append(
                    pltpu.async_copy(
                        rbufs[i], acc_sh.at[ridx_b.at[c0 + i]], ssems[i], add=True
                    )
                )
            for sh in shs:
                sh.wait()

    plsc.subcore_barrier()

    @pl.when(sid == 0)
    def _():
        pltpu.sync_copy(acc_sh, acc_hbm.at[pl.ds(cid * N_PAD, N_PAD)])


BR = 1000


def _dis_from(hist_ref):
    deg = jnp.sum(hist_ref[...], axis=1)
    return jnp.where(deg > 0, lax.rsqrt(deg), 0.0)[:, None]


def _tc_scale_body(hist_ref, x_ref, y_ref):
    y_ref[...] = _dis_from(hist_ref) * x_ref[...]


def _tc_scale(hist_t, x):
    return pl.pallas_call(
        _tc_scale_body,
        grid=(N // BR,),
        in_specs=[
            pl.BlockSpec((BR, NW), lambda i: (i, 0)),
            pl.BlockSpec((BR, D), lambda i: (i, 0)),
        ],
        out_specs=pl.BlockSpec((BR, D), lambda i: (i, 0)),
        out_shape=jax.ShapeDtypeStruct((N, D), jnp.float32),
    )(hist_t, x)


def _tc_finalize_body(hist_ref, x_ref, acc_ref, out_ref):
    out_ref[:, :D] = x_ref[...]
    out_ref[:, D:] = _dis_from(hist_ref) * (acc_ref[0] + acc_ref[1])


def _tc_finalize(hist_t, x, acc2):
    return pl.pallas_call(
        _tc_finalize_body,
        grid=(N // BR,),
        in_specs=[
            pl.BlockSpec((BR, NW), lambda i: (i, 0)),
            pl.BlockSpec((BR, D), lambda i: (i, 0)),
            pl.BlockSpec((2, BR, D), lambda i: (0, i, 0)),
        ],
        out_specs=pl.BlockSpec((BR, 2 * D), lambda i: (i, 0)),
        out_shape=jax.ShapeDtypeStruct((N, 2 * D), jnp.float32),
    )(hist_t, x, acc2)


def kernel(x, edge_index):
    row = edge_index[0]
    col = edge_index[1]
    pad_iota = jnp.arange(E_PAD - E, dtype=jnp.int32)
    pad_r = N + pad_iota % (N_PAD - N)
    pad_c = pad_iota % N
    row2 = jnp.concatenate([row, pad_r]).reshape(NCHUNKS_P, CHUNK)
    col2 = jnp.concatenate([col, pad_c]).reshape(NCHUNKS_P, CHUNK)
    zeros_hist = jnp.zeros((N_PAD,), jnp.float32)
    zeros_acc = jnp.zeros((N_PAD, D), jnp.float32)
    hist_t = _sc_degree(row2, zeros_hist).reshape(NW, N_PAD).T
    y = _tc_scale(hist_t, x)
    acc2 = _sc_aggregate(y, row2, col2, zeros_acc).reshape(NC, N_PAD, D)
    return _tc_finalize(hist_t, x, acc2)
